# baseline (speedup 1.0000x reference)
import functools

import jax
import jax.numpy as jnp
from jax import lax
from jax.experimental import pallas as pl
from jax.experimental.pallas import tpu as pltpu
from jax.experimental.pallas import tpu_sc as plsc

N = 174592
K = 1745
NT = 16
C = N // NT
NV = C // 16
HW = 4096
AW = HW + 16

TBR = 15872
TGB = 11


def _dense_body(cls_ref, cen_ref, sc_ref, w_ref, sum_ref):
    i = pl.program_id(0)
    m = jnp.max(cls_ref[...], axis=0)
    s = jax.nn.sigmoid(m)
    w = s * jax.nn.sigmoid(cen_ref[pl.ds(i * TBR, TBR)])
    sc_ref[pl.ds(i * TBR, TBR)] = s
    w_ref[pl.ds(i * TBR, TBR)] = w
    blk = jnp.sum(s)
    prev = jnp.where(i == 0, 0.0, sum_ref[0, 0])
    acc = prev + blk
    sum_ref[...] = jnp.where(i == TGB - 1, acc / N, acc).reshape(1, 1)


def _dense(cls_t, cen):
    return pl.pallas_call(
        _dense_body,
        grid=(TGB,),
        in_specs=[
            pl.BlockSpec((16, TBR), lambda i: (0, i)),
            pl.BlockSpec((N,), lambda i: (0,)),
        ],
        out_specs=[
            pl.BlockSpec((N,), lambda i: (0,)),
            pl.BlockSpec((N,), lambda i: (0,)),
            pl.BlockSpec((1, 1), lambda i: (0, 0)),
        ],
        out_shape=[
            jax.ShapeDtypeStruct((N,), jnp.float32),
            jax.ShapeDtypeStruct((N,), jnp.float32),
            jax.ShapeDtypeStruct((1, 1), jnp.float32),
        ],
    )(cls_t, cen)


def _lane():
    return lax.iota(jnp.int32, 16)


def _bcast_i(x):
    return jnp.zeros((16,), jnp.int32) + x


def _sel_body(
    scores_hbm,
    mask_hbm,
    stats_hbm,
    scores_v,
    mask_v,
    hist_p,
    hist_n,
    tmp2,
    gh_v,
    a_v,
    gsl,
    ctrl_v,
    fsum_v,
    stats_v,
    sh_hist_p,
    sh_hist_n,
    sh_gh_p,
    sh_gh_n,
    sh_ctrl,
    sh_cnt,
    sh_sum,
):
    tid = lax.axis_index("s")
    lane = _lane()
    lane16 = lane * 16
    rbase = tid * C
    pltpu.sync_copy(scores_hbm.at[pl.ds(rbase, C)], scores_v)

    kpos = jnp.int32(K)
    kneg = jnp.int32(K)
    ppos = jnp.int32(0)
    pneg = jnp.int32(0)

    for pi, sh in enumerate((24, 16, 8, 0)):

        def zbody(j, _):
            z = jnp.zeros((16,), jnp.int32)
            hist_p[pl.ds(j * 16, 16)] = z
            if pi > 0:
                hist_n[pl.ds(j * 16, 16)] = z
            return 0

        lax.fori_loop(0, 256, zbody, 0, unroll=8)

        ppos_v = _bcast_i(ppos)
        pneg_v = _bcast_i(pneg)
        one = jnp.ones((16,), jnp.int32)

        if pi == 0:

            def sbody(i, _):
                k = plsc.bitcast(scores_v[pl.ds(i * 16, 16)], jnp.int32)
                d = (jnp.right_shift(k, 24) + 128) & 255
                idx = d * 16 + lane
                plsc.addupdate_scatter(hist_p, [idx], one)
                return 0

        else:

            def sbody(i, _):
                k = plsc.bitcast(scores_v[pl.ds(i * 16, 16)], jnp.int32)
                d = jnp.right_shift(k, sh) & 255
                hi = jnp.right_shift(k, sh + 8)
                idx = d * 16 + lane
                plsc.addupdate_scatter(hist_p, [idx], one, mask=hi == ppos_v)
                plsc.addupdate_scatter(hist_n, [idx], one, mask=hi == pneg_v)
                return 0

        lax.fori_loop(0, NV, sbody, 0, unroll=4)

        pltpu.sync_copy(hist_p, sh_hist_p.at[tid])
        if pi > 0:
            pltpu.sync_copy(hist_n, sh_hist_n.at[tid])
        plsc.subcore_barrier()

        sides = (
            ((sh_hist_p, sh_gh_p),)
            if pi == 0
            else ((sh_hist_p, sh_gh_p), (sh_hist_n, sh_gh_n))
        )
        for sh_hist, sh_gh in sides:
            for r in range(NT):
                pltpu.sync_copy(
                    sh_hist.at[r, pl.ds(tid * 256, 256)],
                    tmp2.at[pl.ds(r * 256, 256)],
                )

            def jbody(j, _):
                acc = jnp.zeros((16,), jnp.int32)
                for r in range(NT):
                    acc = acc + tmp2[pl.ds(r * 256 + j * 16, 16)]
                gsl[pl.ds(j * 16, 16)] = acc
                return 0

            lax.fori_loop(0, 16, jbody, 0, unroll=2)
            pltpu.sync_copy(gsl, sh_gh.at[pl.ds(tid * 256, 256)])
        plsc.subcore_barrier()

        @pl.when(tid == 0)
        def _():
            def prefix_and_total(_):
                def pbody(j, carry):
                    gv = gh_v[pl.ds(j * 16, 16)]
                    cs = plsc.cumsum(gv)
                    a_v[pl.ds(j * 16, 16)] = cs - gv + carry
                    return carry + jnp.sum(gv)

                total = lax.fori_loop(0, 256, pbody, jnp.int32(0), unroll=4)
                a_v[pl.ds(HW, 16)] = _bcast_i(total)
                return total

            pltpu.sync_copy(sh_gh_p, gh_v)
            total_p = prefix_and_total(None)
            target = total_p - kpos

            def cbody(j, cnt):
                bvec = j * 16 + lane
                e = plsc.load_gather(a_v, [bvec * 16])
                return cnt + jnp.sum(jnp.where(e <= target, 1, 0))

            bstar = lax.fori_loop(0, 16, cbody, jnp.int32(0), unroll=4) - 1
            eb1 = jnp.max(plsc.load_gather(a_v, [_bcast_i((bstar + 1) * 16)]))
            kpos_n = kpos - (total_p - eb1)
            ppos_n = bstar - 128 if pi == 0 else ppos * 256 + bstar

            if pi > 0:
                pltpu.sync_copy(sh_gh_n, gh_v)
                prefix_and_total(None)

            def nbody(j, cnt):
                bvec = j * 16 + lane
                f = plsc.load_gather(a_v, [(bvec + 1) * 16])
                return cnt + jnp.sum(jnp.where(f < kneg, 1, 0))

            bn = lax.fori_loop(0, 16, nbody, jnp.int32(0), unroll=4)
            ebn = jnp.max(plsc.load_gather(a_v, [_bcast_i(bn * 16)]))
            kneg_n = kneg - ebn
            pneg_n = bn - 128 if pi == 0 else pneg * 256 + bn

            ctrl = (
                jnp.where(lane == 0, ppos_n, 0)
                + jnp.where(lane == 1, kpos_n, 0)
                + jnp.where(lane == 2, pneg_n, 0)
                + jnp.where(lane == 3, kneg_n, 0)
            )
            ctrl_v[...] = ctrl
            pltpu.sync_copy(ctrl_v, sh_ctrl)

        plsc.subcore_barrier()
        pltpu.sync_copy(sh_ctrl, ctrl_v)
        cv = ctrl_v[...]
        ppos = jnp.sum(jnp.where(lane == 0, cv, 0))
        kpos = jnp.sum(jnp.where(lane == 1, cv, 0))
        pneg = jnp.sum(jnp.where(lane == 2, cv, 0))
        kneg = jnp.sum(jnp.where(lane == 3, cv, 0))

    tpos_v = _bcast_i(ppos)
    tneg_v = _bcast_i(pneg)

    def stbody(i, carry):
        ep, en, sg = carry
        s = scores_v[pl.ds(i * 16, 16)]
        k = plsc.bitcast(s, jnp.int32)
        ep = ep + jnp.sum(jnp.where(k == tpos_v, 1, 0))
        en = en + jnp.sum(jnp.where(k == tneg_v, 1, 0))
        sg = sg + jnp.sum(jnp.where(k > tpos_v, s, 0.0))
        return ep, en, sg

    ep, en, sg = lax.fori_loop(
        0, NV, stbody, (jnp.int32(0), jnp.int32(0), jnp.float32(0.0)), unroll=4
    )
    ctrl_v[...] = jnp.where(lane == 0, ep, 0) + jnp.where(lane == 1, en, 0)
    pltpu.sync_copy(ctrl_v, sh_cnt.at[pl.ds(tid * 16, 16)])
    stats_v[...] = jnp.where(lane == 0, sg, 0.0)
    pltpu.sync_copy(stats_v, sh_sum.at[pl.ds(tid * 16, 16)])
    plsc.subcore_barrier()

    pltpu.sync_copy(sh_cnt, tmp2.at[pl.ds(0, 256)])
    pltpu.sync_copy(sh_sum, fsum_v)
    eqp_all = plsc.load_gather(tmp2, [lane16])
    eqn_all = plsc.load_gather(tmp2, [lane16 + 1])
    sg_all = plsc.load_gather(fsum_v, [lane16])
    pre_p = jnp.sum(jnp.where(lane < tid, eqp_all, 0))
    pre_n = jnp.sum(jnp.where(lane < tid, eqn_all, 0))
    sg_tot = jnp.sum(sg_all)

    @pl.when(tid == 0)
    def _():
        s_thr = jnp.max(plsc.bitcast(tpos_v, jnp.float32))
        fg = sg_tot + kpos.astype(jnp.float32) * s_thr
        stats_v[...] = jnp.where(lane == 0, fg, 0.0)
        pltpu.sync_copy(stats_v, stats_hbm)

    def mbody(i, carry):
        cp, cn = carry
        k = plsc.bitcast(scores_v[pl.ds(i * 16, 16)], jnp.int32)
        eqp = k == tpos_v
        ei = jnp.where(eqp, 1, 0)
        csp = plsc.cumsum(ei)
        takep = eqp & (cp + csp - 1 < kpos)
        eqn = k == tneg_v
        ni = jnp.where(eqn, 1, 0)
        csn = plsc.cumsum(ni)
        taken = eqn & (cn + csn - 1 < kneg)
        m = jnp.where(
            (k < tneg_v) | taken,
            -1.0,
            jnp.where((k > tpos_v) | takep, 1.0, 0.0),
        )
        mask_v[pl.ds(i * 16, 16)] = m
        return cp + jnp.sum(ei), cn + jnp.sum(ni)

    lax.fori_loop(0, NV, mbody, (pre_p, pre_n), unroll=2)
    pltpu.sync_copy(mask_v, mask_hbm.at[pl.ds(rbase, C)])


@functools.cache
def _get_select():
    return pl.kernel(
        _sel_body,
        out_type=[
            jax.ShapeDtypeStruct((N,), jnp.float32),
            jax.ShapeDtypeStruct((16,), jnp.float32),
        ],
        mesh=plsc.VectorSubcoreMesh(
            core_axis_name="c", subcore_axis_name="s", num_cores=1
        ),
        compiler_params=pltpu.CompilerParams(needs_layout_passes=False),
        scratch_types=[
            pltpu.VMEM((C,), jnp.float32),
            pltpu.VMEM((C,), jnp.float32),
            pltpu.VMEM((HW,), jnp.int32),
            pltpu.VMEM((HW,), jnp.int32),
            pltpu.VMEM((HW,), jnp.int32),
            pltpu.VMEM((HW,), jnp.int32),
            pltpu.VMEM((AW,), jnp.int32),
            pltpu.VMEM((256,), jnp.int32),
            pltpu.VMEM((16,), jnp.int32),
            pltpu.VMEM((256,), jnp.float32),
            pltpu.VMEM((16,), jnp.float32),
            pltpu.VMEM_SHARED((16, HW), jnp.int32),
            pltpu.VMEM_SHARED((16, HW), jnp.int32),
            pltpu.VMEM_SHARED((HW,), jnp.int32),
            pltpu.VMEM_SHARED((HW,), jnp.int32),
            pltpu.VMEM_SHARED((16,), jnp.int32),
            pltpu.VMEM_SHARED((256,), jnp.int32),
            pltpu.VMEM_SHARED((256,), jnp.float32),
        ],
    )


def kernel(t_cls_scores, t_bbox_preds, t_centernesses):
    scores, weight, sdps = _dense(t_cls_scores.T, t_centernesses.reshape(N))
    mask, stats = _get_select()(scores)
    pos_mask = mask > 0.0
    neg_mask = mask < 0.0
    return pos_mask, neg_mask, weight, stats[0], sdps[0, 0]


# speedup vs baseline: 1.1118x; 1.1118x over previous
import functools

import jax
import jax.numpy as jnp
from jax import lax
from jax.experimental import pallas as pl
from jax.experimental.pallas import tpu as pltpu
from jax.experimental.pallas import tpu_sc as plsc

N = 174592
K = 1745
NT = 16
C = N // NT
NV = C // 16
HW = 4096
AW = HW + 16

TBR = 15872
TGB = 11


def _dense_body(cls_ref, cen_ref, sc_ref, w_ref, sum_ref):
    i = pl.program_id(0)
    m = jnp.max(cls_ref[...], axis=0)
    s = jax.nn.sigmoid(m)
    w = s * jax.nn.sigmoid(cen_ref[pl.ds(i * TBR, TBR)])
    sc_ref[pl.ds(i * TBR, TBR)] = s
    w_ref[pl.ds(i * TBR, TBR)] = w
    blk = jnp.sum(s)
    prev = jnp.where(i == 0, 0.0, sum_ref[0, 0])
    acc = prev + blk
    sum_ref[...] = jnp.where(i == TGB - 1, acc / N, acc).reshape(1, 1)


def _dense(cls_t, cen):
    return pl.pallas_call(
        _dense_body,
        grid=(TGB,),
        in_specs=[
            pl.BlockSpec((16, TBR), lambda i: (0, i)),
            pl.BlockSpec((N,), lambda i: (0,)),
        ],
        out_specs=[
            pl.BlockSpec((N,), lambda i: (0,)),
            pl.BlockSpec((N,), lambda i: (0,)),
            pl.BlockSpec((1, 1), lambda i: (0, 0)),
        ],
        out_shape=[
            jax.ShapeDtypeStruct((N,), jnp.float32),
            jax.ShapeDtypeStruct((N,), jnp.float32),
            jax.ShapeDtypeStruct((1, 1), jnp.float32),
        ],
    )(cls_t, cen)


def _lane():
    return lax.iota(jnp.int32, 16)


def _bcast_i(x):
    return jnp.zeros((16,), jnp.int32) + x


def _sel_body(
    scores_hbm,
    mask_hbm,
    stats_hbm,
    scores_v,
    mask_v,
    hist_p,
    hist_n,
    tmp2,
    gh_v,
    a_v,
    gsl,
    ctrl_v,
    fsum_v,
    stats_v,
    sh_hist_p,
    sh_hist_n,
    sh_gh_p,
    sh_gh_n,
    sh_ctrl,
    sh_cnt,
    sh_sum,
):
    tid = lax.axis_index("s")
    lane = _lane()
    lane16 = lane * 16
    rbase = tid * C
    pltpu.sync_copy(scores_hbm.at[pl.ds(rbase, C)], scores_v)

    kpos = jnp.int32(K)
    kneg = jnp.int32(K)
    ppos = jnp.int32(0)
    pneg = jnp.int32(0)

    for pi, sh in enumerate((24, 16, 8, 0)):

        def zbody(j, _):
            z = jnp.zeros((16,), jnp.int32)
            hist_p[pl.ds(j * 16, 16)] = z
            if pi > 0:
                hist_n[pl.ds(j * 16, 16)] = z
            return 0

        lax.fori_loop(0, 256, zbody, 0)

        ppos_v = _bcast_i(ppos)
        pneg_v = _bcast_i(pneg)
        one = jnp.ones((16,), jnp.int32)

        if pi == 0:

            def sbody(i, _):
                k = plsc.bitcast(scores_v[pl.ds(i * 16, 16)], jnp.int32)
                d = (jnp.right_shift(k, 24) + 128) & 255
                idx = d * 16 + lane
                plsc.addupdate_scatter(hist_p, [idx], one)
                return 0

        else:

            def sbody(i, _):
                k = plsc.bitcast(scores_v[pl.ds(i * 16, 16)], jnp.int32)
                d = jnp.right_shift(k, sh) & 255
                hi = jnp.right_shift(k, sh + 8)
                idx = d * 16 + lane
                plsc.addupdate_scatter(hist_p, [idx], one, mask=hi == ppos_v)
                plsc.addupdate_scatter(hist_n, [idx], one, mask=hi == pneg_v)
                return 0

        lax.fori_loop(0, NV, sbody, 0)

        pltpu.sync_copy(hist_p, sh_hist_p.at[tid])
        if pi > 0:
            pltpu.sync_copy(hist_n, sh_hist_n.at[tid])
        plsc.subcore_barrier()

        sides = (
            ((sh_hist_p, sh_gh_p),)
            if pi == 0
            else ((sh_hist_p, sh_gh_p), (sh_hist_n, sh_gh_n))
        )
        for sh_hist, sh_gh in sides:
            for r in range(NT):
                pltpu.sync_copy(
                    sh_hist.at[r, pl.ds(tid * 256, 256)],
                    tmp2.at[pl.ds(r * 256, 256)],
                )

            def jbody(j, _):
                acc = jnp.zeros((16,), jnp.int32)
                for r in range(NT):
                    acc = acc + tmp2[pl.ds(r * 256 + j * 16, 16)]
                gsl[pl.ds(j * 16, 16)] = acc
                return 0

            lax.fori_loop(0, 16, jbody, 0)
            pltpu.sync_copy(gsl, sh_gh.at[pl.ds(tid * 256, 256)])
        plsc.subcore_barrier()

        @pl.when(tid == 0)
        def _():
            def prefix_and_total(_):
                def pbody(j, carry):
                    gv = gh_v[pl.ds(j * 16, 16)]
                    cs = plsc.cumsum(gv)
                    a_v[pl.ds(j * 16, 16)] = cs - gv + carry
                    return carry + jnp.sum(gv)

                total = lax.fori_loop(0, 256, pbody, jnp.int32(0))
                a_v[pl.ds(HW, 16)] = _bcast_i(total)
                return total

            pltpu.sync_copy(sh_gh_p, gh_v)
            total_p = prefix_and_total(None)
            target = total_p - kpos

            def cbody(j, cnt):
                bvec = j * 16 + lane
                e = plsc.load_gather(a_v, [bvec * 16])
                return cnt + jnp.sum(jnp.where(e <= target, 1, 0))

            bstar = lax.fori_loop(0, 16, cbody, jnp.int32(0)) - 1
            eb1 = jnp.max(plsc.load_gather(a_v, [_bcast_i((bstar + 1) * 16)]))
            kpos_n = kpos - (total_p - eb1)
            ppos_n = bstar - 128 if pi == 0 else ppos * 256 + bstar

            if pi > 0:
                pltpu.sync_copy(sh_gh_n, gh_v)
                prefix_and_total(None)

            def nbody(j, cnt):
                bvec = j * 16 + lane
                f = plsc.load_gather(a_v, [(bvec + 1) * 16])
                return cnt + jnp.sum(jnp.where(f < kneg, 1, 0))

            bn = lax.fori_loop(0, 16, nbody, jnp.int32(0))
            ebn = jnp.max(plsc.load_gather(a_v, [_bcast_i(bn * 16)]))
            kneg_n = kneg - ebn
            pneg_n = bn - 128 if pi == 0 else pneg * 256 + bn

            ctrl = (
                jnp.where(lane == 0, ppos_n, 0)
                + jnp.where(lane == 1, kpos_n, 0)
                + jnp.where(lane == 2, pneg_n, 0)
                + jnp.where(lane == 3, kneg_n, 0)
            )
            ctrl_v[...] = ctrl
            pltpu.sync_copy(ctrl_v, sh_ctrl)

        plsc.subcore_barrier()
        pltpu.sync_copy(sh_ctrl, ctrl_v)
        cv = ctrl_v[...]
        ppos = jnp.sum(jnp.where(lane == 0, cv, 0))
        kpos = jnp.sum(jnp.where(lane == 1, cv, 0))
        pneg = jnp.sum(jnp.where(lane == 2, cv, 0))
        kneg = jnp.sum(jnp.where(lane == 3, cv, 0))

    tpos_v = _bcast_i(ppos)
    tneg_v = _bcast_i(pneg)

    def stbody(i, carry):
        ep, en, sg = carry
        s = scores_v[pl.ds(i * 16, 16)]
        k = plsc.bitcast(s, jnp.int32)
        ep = ep + jnp.sum(jnp.where(k == tpos_v, 1, 0))
        en = en + jnp.sum(jnp.where(k == tneg_v, 1, 0))
        sg = sg + jnp.sum(jnp.where(k > tpos_v, s, 0.0))
        return ep, en, sg

    ep, en, sg = lax.fori_loop(
        0, NV, stbody, (jnp.int32(0), jnp.int32(0), jnp.float32(0.0))
    )
    ctrl_v[...] = jnp.where(lane == 0, ep, 0) + jnp.where(lane == 1, en, 0)
    pltpu.sync_copy(ctrl_v, sh_cnt.at[pl.ds(tid * 16, 16)])
    stats_v[...] = jnp.where(lane == 0, sg, 0.0)
    pltpu.sync_copy(stats_v, sh_sum.at[pl.ds(tid * 16, 16)])
    plsc.subcore_barrier()

    pltpu.sync_copy(sh_cnt, tmp2.at[pl.ds(0, 256)])
    pltpu.sync_copy(sh_sum, fsum_v)
    eqp_all = plsc.load_gather(tmp2, [lane16])
    eqn_all = plsc.load_gather(tmp2, [lane16 + 1])
    sg_all = plsc.load_gather(fsum_v, [lane16])
    pre_p = jnp.sum(jnp.where(lane < tid, eqp_all, 0))
    pre_n = jnp.sum(jnp.where(lane < tid, eqn_all, 0))
    sg_tot = jnp.sum(sg_all)

    @pl.when(tid == 0)
    def _():
        s_thr = jnp.max(plsc.bitcast(tpos_v, jnp.float32))
        fg = sg_tot + kpos.astype(jnp.float32) * s_thr
        stats_v[...] = jnp.where(lane == 0, fg, 0.0)
        pltpu.sync_copy(stats_v, stats_hbm)

    def mbody(i, carry):
        cp, cn = carry
        k = plsc.bitcast(scores_v[pl.ds(i * 16, 16)], jnp.int32)
        eqp = k == tpos_v
        ei = jnp.where(eqp, 1, 0)
        csp = plsc.cumsum(ei)
        takep = eqp & (cp + csp - 1 < kpos)
        eqn = k == tneg_v
        ni = jnp.where(eqn, 1, 0)
        csn = plsc.cumsum(ni)
        taken = eqn & (cn + csn - 1 < kneg)
        m = jnp.where(
            (k < tneg_v) | taken,
            -1.0,
            jnp.where((k > tpos_v) | takep, 1.0, 0.0),
        )
        mask_v[pl.ds(i * 16, 16)] = m
        return cp + jnp.sum(ei), cn + jnp.sum(ni)

    lax.fori_loop(0, NV, mbody, (pre_p, pre_n))
    pltpu.sync_copy(mask_v, mask_hbm.at[pl.ds(rbase, C)])


@functools.cache
def _get_select():
    return pl.kernel(
        _sel_body,
        out_type=[
            jax.ShapeDtypeStruct((N,), jnp.float32),
            jax.ShapeDtypeStruct((16,), jnp.float32),
        ],
        mesh=plsc.VectorSubcoreMesh(
            core_axis_name="c", subcore_axis_name="s", num_cores=1
        ),
        compiler_params=pltpu.CompilerParams(needs_layout_passes=False),
        scratch_types=[
            pltpu.VMEM((C,), jnp.float32),
            pltpu.VMEM((C,), jnp.float32),
            pltpu.VMEM((HW,), jnp.int32),
            pltpu.VMEM((HW,), jnp.int32),
            pltpu.VMEM((HW,), jnp.int32),
            pltpu.VMEM((HW,), jnp.int32),
            pltpu.VMEM((AW,), jnp.int32),
            pltpu.VMEM((256,), jnp.int32),
            pltpu.VMEM((16,), jnp.int32),
            pltpu.VMEM((256,), jnp.float32),
            pltpu.VMEM((16,), jnp.float32),
            pltpu.VMEM_SHARED((16, HW), jnp.int32),
            pltpu.VMEM_SHARED((16, HW), jnp.int32),
            pltpu.VMEM_SHARED((HW,), jnp.int32),
            pltpu.VMEM_SHARED((HW,), jnp.int32),
            pltpu.VMEM_SHARED((16,), jnp.int32),
            pltpu.VMEM_SHARED((256,), jnp.int32),
            pltpu.VMEM_SHARED((256,), jnp.float32),
        ],
    )


def kernel(t_cls_scores, t_bbox_preds, t_centernesses):
    scores, weight, sdps = _dense(t_cls_scores.T, t_centernesses.reshape(N))
    mask, stats = _get_select()(scores)
    pos_mask = mask > 0.0
    neg_mask = mask < 0.0
    return pos_mask, neg_mask, weight, stats[0], sdps[0, 0]
